# Initial kernel scaffold; baseline (speedup 1.0000x reference)
import functools
import jax
import jax.numpy as jnp
from jax.experimental import pallas as pl
from jax.experimental.pallas import tpu as pltpu

N = 1_000_000
D = 64
Q = 32
K = 10
BLK = 128
TILE = 16384
NT = (N + TILE - 1) // TILE
NBPT = TILE // BLK
NB = NT * NBPT
NEG = jnp.float32(-jnp.inf)


def _k1_blockmax(q_ref, bank_ref, bm_ref):
    t = pl.program_id(0)
    sim = jax.lax.dot_general(
        q_ref[...], bank_ref[...], (((1,), (1,)), ((), ())),
        preferred_element_type=jnp.float32)
    rows = jax.lax.broadcasted_iota(jnp.int32, (Q, TILE), 1) + t * TILE
    sim = jnp.where(rows < N, sim, NEG)
    bm_ref[...] = jnp.max(sim.reshape(Q, NBPT, BLK), axis=2)


def _k2_select(bm_ref, logits_ref, idx_ref, conf_ref):
    bm = bm_ref[...]
    iot = jax.lax.broadcasted_iota(jnp.int32, (Q, NB), 1)
    cols = []
    for _ in range(K):
        m = jnp.max(bm, axis=1, keepdims=True)
        amin = jnp.min(jnp.where(bm == m, iot, NB), axis=1, keepdims=True)
        cols.append(amin)
        bm = jnp.where(iot == amin, NEG, bm)
    cols += [jnp.zeros((Q, 1), jnp.int32)] * (16 - K)
    idx_ref[...] = jnp.concatenate(cols, axis=1)
    lg = logits_ref[...]
    mx = jnp.max(lg, axis=1, keepdims=True)
    s = jnp.sum(jnp.exp(lg - mx), axis=1, keepdims=True)
    conf_ref[...] = jnp.log(s) + mx


def _k3_rescore(idx_sref, bank_ref, feat_ref, conf_ref, out_ref, simbuf):
    q = pl.program_id(0)
    j = pl.program_id(1)
    sims = jax.lax.dot_general(
        feat_ref[...], bank_ref[...], (((1,), (1,)), ((), ())),
        preferred_element_type=jnp.float32)
    blk = idx_sref[q, j]
    rows = jax.lax.broadcasted_iota(jnp.int32, (1, BLK), 1) + blk * BLK
    simbuf[0, pl.ds(j * BLK, BLK)] = jnp.where(rows < N, sims, NEG)[0]

    @pl.when(j == K - 1)
    def _():
        s = simbuf[...].reshape(1, K * BLK)
        iot = jax.lax.broadcasted_iota(jnp.int32, (1, K * BLK), 1)
        total = jnp.float32(0.0)
        for _ in range(K):
            m = jnp.max(s, axis=1, keepdims=True)
            total = total + m[0, 0]
            amin = jnp.min(jnp.where(s == m, iot, K * BLK), axis=1,
                           keepdims=True)
            s = jnp.where(iot == amin, NEG, s)
        score = total / K * conf_ref[0, 0]
        out_ref[...] = jnp.full((1, 128), score, jnp.float32)


@jax.jit
def kernel(logits, features, scaled_feas):
    bm = pl.pallas_call(
        _k1_blockmax,
        grid=(NT,),
        in_specs=[
            pl.BlockSpec((Q, D), lambda i: (0, 0)),
            pl.BlockSpec((TILE, D), lambda i: (i, 0)),
        ],
        out_specs=pl.BlockSpec((Q, NBPT), lambda i: (0, i)),
        out_shape=jax.ShapeDtypeStruct((Q, NB), jnp.float32),
    )(features, scaled_feas)

    idx, conf = pl.pallas_call(
        _k2_select,
        in_specs=[
            pl.BlockSpec((Q, NB), lambda: (0, 0)),
            pl.BlockSpec(logits.shape, lambda: (0, 0)),
        ],
        out_specs=[
            pl.BlockSpec((Q, 16), lambda: (0, 0)),
            pl.BlockSpec((Q, 1), lambda: (0, 0)),
        ],
        out_shape=[
            jax.ShapeDtypeStruct((Q, 16), jnp.int32),
            jax.ShapeDtypeStruct((Q, 1), jnp.float32),
        ],
    )(bm, logits)

    out = pl.pallas_call(
        _k3_rescore,
        grid_spec=pltpu.PrefetchScalarGridSpec(
            num_scalar_prefetch=1,
            grid=(Q, K),
            in_specs=[
                pl.BlockSpec((BLK, D), lambda q, j, idx: (idx[q, j], 0)),
                pl.BlockSpec((1, D), lambda q, j, idx: (q, 0)),
                pl.BlockSpec((1, 1), lambda q, j, idx: (q, 0)),
            ],
            out_specs=pl.BlockSpec((1, 128), lambda q, j, idx: (q, 0)),
            scratch_shapes=[pltpu.VMEM((1, K * BLK), jnp.float32)],
        ),
        out_shape=jax.ShapeDtypeStruct((Q, 128), jnp.float32),
    )(idx, scaled_feas, features, conf)

    return out[:, 0]


# speedup vs baseline: 1.1565x; 1.1565x over previous
import functools
import jax
import jax.numpy as jnp
from jax.experimental import pallas as pl
from jax.experimental.pallas import tpu as pltpu

N = 1_000_000
D = 64
Q = 32
K = 10
BLK = 128
TILE = 16384
NT = (N + TILE - 1) // TILE
NBPT = TILE // BLK
NB = NT * NBPT
NEG = float("-inf")


def _k1_blockmax(q_ref, bank_ref, bm_ref):
    t = pl.program_id(0)
    sim = jax.lax.dot_general(
        q_ref[...], bank_ref[...], (((1,), (1,)), ((), ())),
        preferred_element_type=jnp.float32)
    rows = jax.lax.broadcasted_iota(jnp.int32, (Q, TILE), 1) + t * TILE
    sim = jnp.where(rows < N, sim, NEG)
    bm_ref[...] = jnp.max(sim.reshape(Q, NBPT, BLK), axis=2)


def _k2_select(bm_ref, logits_ref, idx_ref, conf_ref):
    bm = bm_ref[...]
    iot = jax.lax.broadcasted_iota(jnp.int32, (Q, NB), 1)
    cols = []
    for _ in range(K):
        m = jnp.max(bm, axis=1, keepdims=True)
        amin = jnp.min(jnp.where(bm == m, iot, NB), axis=1, keepdims=True)
        cols.append(amin)
        bm = jnp.where(iot == amin, NEG, bm)
    cols += [jnp.zeros((Q, 1), jnp.int32)] * (16 - K)
    idx_ref[...] = jnp.concatenate(cols, axis=1)
    lg = logits_ref[...]
    mx = jnp.max(lg, axis=1, keepdims=True)
    s = jnp.sum(jnp.exp(lg - mx), axis=1, keepdims=True)
    conf_ref[...] = jnp.log(s) + mx


def _k3_rescore(idx_sref, bank_ref, feat_ref, conf_ref, out_ref, simbuf):
    q = pl.program_id(0)
    j = pl.program_id(1)
    feat = feat_ref[pl.ds(q, 1), :]
    sims = jax.lax.dot_general(
        feat, bank_ref[...], (((1,), (1,)), ((), ())),
        preferred_element_type=jnp.float32)
    blk = idx_sref[q, j]
    rows = jax.lax.broadcasted_iota(jnp.int32, (1, BLK), 1) + blk * BLK
    simbuf[0, pl.ds(j * BLK, BLK)] = jnp.where(rows < N, sims, NEG)[0]

    @pl.when(j == K - 1)
    def _():
        s = simbuf[...].reshape(1, K * BLK)
        iot = jax.lax.broadcasted_iota(jnp.int32, (1, K * BLK), 1)
        total = jnp.float32(0.0)
        for _ in range(K):
            m = jnp.max(s, axis=1, keepdims=True)
            total = total + m[0, 0]
            amin = jnp.min(jnp.where(s == m, iot, K * BLK), axis=1,
                           keepdims=True)
            s = jnp.where(iot == amin, NEG, s)
        score = total / K * conf_ref[pl.ds(q, 1), :][0, 0]
        out_ref[pl.ds(q, 1), :] = jnp.full((1, 128), score, jnp.float32)


@jax.jit
def kernel(logits, features, scaled_feas):
    bm = pl.pallas_call(
        _k1_blockmax,
        grid=(NT,),
        in_specs=[
            pl.BlockSpec((Q, D), lambda i: (0, 0)),
            pl.BlockSpec((TILE, D), lambda i: (i, 0)),
        ],
        out_specs=pl.BlockSpec((Q, NBPT), lambda i: (0, i)),
        out_shape=jax.ShapeDtypeStruct((Q, NB), jnp.float32),
    )(features, scaled_feas)

    idx, conf = pl.pallas_call(
        _k2_select,
        in_specs=[
            pl.BlockSpec((Q, NB), lambda: (0, 0)),
            pl.BlockSpec(logits.shape, lambda: (0, 0)),
        ],
        out_specs=[
            pl.BlockSpec((Q, 16), lambda: (0, 0)),
            pl.BlockSpec((Q, 1), lambda: (0, 0)),
        ],
        out_shape=[
            jax.ShapeDtypeStruct((Q, 16), jnp.int32),
            jax.ShapeDtypeStruct((Q, 1), jnp.float32),
        ],
    )(bm, logits)

    out = pl.pallas_call(
        _k3_rescore,
        grid_spec=pltpu.PrefetchScalarGridSpec(
            num_scalar_prefetch=1,
            grid=(Q, K),
            in_specs=[
                pl.BlockSpec((BLK, D), lambda q, j, idx: (idx[q, j], 0)),
                pl.BlockSpec((Q, D), lambda q, j, idx: (0, 0)),
                pl.BlockSpec((Q, 1), lambda q, j, idx: (0, 0)),
            ],
            out_specs=pl.BlockSpec((Q, 128), lambda q, j, idx: (0, 0)),
            scratch_shapes=[pltpu.VMEM((1, K * BLK), jnp.float32)],
        ),
        out_shape=jax.ShapeDtypeStruct((Q, 128), jnp.float32),
    )(idx, scaled_feas, features, conf)

    return out[:, 0]
